# Initial kernel scaffold; baseline (speedup 1.0000x reference)
import functools

import jax
import jax.numpy as jnp
from jax import lax
from jax.experimental import pallas as pl
from jax.experimental.pallas import tpu as pltpu
from jax.experimental.pallas import tpu_sc as plsc

_NC = 2
_NS = 16
_NW = _NC * _NS


def _sc_gather(x, src):
    n, node_in = x.shape
    e = src.shape[0]
    e_per_w = e // _NW
    mesh = plsc.VectorSubcoreMesh(core_axis_name="c", subcore_axis_name="s")

    @functools.partial(
        pl.kernel,
        mesh=mesh,
        out_type=jax.ShapeDtypeStruct((e, node_in), jnp.float32),
        scratch_types=[
            pltpu.VMEM((e_per_w,), jnp.int32),
            pltpu.VMEM((e_per_w, node_in), jnp.float32),
            pltpu.SemaphoreType.DMA,
        ],
    )
    def k(x_hbm, src_hbm, out_hbm, idx_v, rows_v, sem):
        wid = lax.axis_index("s") * _NC + lax.axis_index("c")
        base = wid * e_per_w
        pltpu.sync_copy(src_hbm.at[pl.ds(base, e_per_w)], idx_v)
        pltpu.async_copy(x_hbm.at[idx_v], rows_v, sem).wait()
        pltpu.sync_copy(rows_v, out_hbm.at[pl.ds(base, e_per_w)])

    return k(x, src)


def _tc_dense(edge_attr, xj, w1, b1r, w2p, b2r, block_e):
    e, node_in = edge_attr.shape
    hidden = w2p.shape[1]
    grid = e // block_e

    def body(ea_ref, xj_ref, w1_ref, b1_ref, w2p_ref, b2r_ref, out_ref):
        ea = ea_ref[...]
        xj = xj_ref[...]
        h = jnp.dot(ea, w1_ref[...], preferred_element_type=jnp.float32)
        h = jnp.maximum(h + b1_ref[...], 0.0)
        z = jnp.concatenate(
            [xj[:, i : i + 1] * h for i in range(node_in)], axis=1
        )
        msg = jnp.dot(z, w2p_ref[...], preferred_element_type=jnp.float32)
        msg = msg + jnp.dot(xj, b2r_ref[...], preferred_element_type=jnp.float32)
        out_ref[...] = msg

    return pl.pallas_call(
        body,
        grid=(grid,),
        in_specs=[
            pl.BlockSpec((block_e, node_in), lambda i: (i, 0)),
            pl.BlockSpec((block_e, node_in), lambda i: (i, 0)),
            pl.BlockSpec(w1.shape, lambda i: (0, 0)),
            pl.BlockSpec(b1r.shape, lambda i: (0, 0)),
            pl.BlockSpec(w2p.shape, lambda i: (0, 0)),
            pl.BlockSpec(b2r.shape, lambda i: (0, 0)),
        ],
        out_specs=pl.BlockSpec((block_e, hidden), lambda i: (i, 0)),
        out_shape=jax.ShapeDtypeStruct((e, hidden), jnp.float32),
    )(edge_attr, xj, w1, b1r, w2p, b2r)


def _sc_scatter(msg, dst, n):
    e, hidden = msg.shape
    e_per_w = e // _NW
    chunk = 1000
    n_chunks = e_per_w // chunk
    rows_per_t = n // _NS
    mesh = plsc.VectorSubcoreMesh(core_axis_name="c", subcore_axis_name="s")

    @functools.partial(
        pl.kernel,
        mesh=mesh,
        out_type=(
            jax.ShapeDtypeStruct((_NC * n, hidden), jnp.float32),
            jax.ShapeDtypeStruct((_NC * n, 16), jnp.float32),
        ),
        scratch_types=[
            pltpu.VMEM((chunk, hidden), jnp.float32),
            pltpu.VMEM((chunk,), jnp.int32),
            pltpu.VMEM((chunk, 16), jnp.float32),
            pltpu.VMEM((rows_per_t, hidden), jnp.float32),
            pltpu.VMEM((rows_per_t, 16), jnp.float32),
            pltpu.VMEM_SHARED((n, hidden), jnp.float32),
            pltpu.VMEM_SHARED((n, 16), jnp.float32),
        ],
    )
    def k(msg_hbm, dst_hbm, s_out, c_out, msg_v, idx_v, ones_v, zh_v, zc_v,
          table_sh, cnt_sh):
        cid = lax.axis_index("c")
        sid = lax.axis_index("s")
        wid = sid * _NC + cid

        zeros16 = jnp.zeros((16,), jnp.float32)
        ones16 = jnp.ones((16,), jnp.float32)

        def fill_ones(j, _):
            ones_v[j, pl.ds(0, 16)] = ones16
            return 0

        lax.fori_loop(0, chunk, fill_ones, 0)

        per_row = hidden // 16

        def zero_h(j, _):
            zh_v[j // per_row, pl.ds((j % per_row) * 16, 16)] = zeros16
            return 0

        lax.fori_loop(0, rows_per_t * per_row, zero_h, 0)

        def zero_c(j, _):
            zc_v[j, pl.ds(0, 16)] = zeros16
            return 0

        lax.fori_loop(0, rows_per_t, zero_c, 0)

        pltpu.sync_copy(zh_v, table_sh.at[pl.ds(sid * rows_per_t, rows_per_t)])
        pltpu.sync_copy(zc_v, cnt_sh.at[pl.ds(sid * rows_per_t, rows_per_t)])
        plsc.subcore_barrier()

        base = wid * e_per_w

        def chunk_body(ci, _):
            off = base + ci * chunk
            pltpu.sync_copy(msg_hbm.at[pl.ds(off, chunk)], msg_v)
            pltpu.sync_copy(dst_hbm.at[pl.ds(off, chunk)], idx_v)
            pltpu.sync_copy(msg_v, table_sh.at[idx_v], add=True)
            pltpu.sync_copy(ones_v, cnt_sh.at[idx_v], add=True)
            return 0

        lax.fori_loop(0, n_chunks, chunk_body, 0)
        plsc.subcore_barrier()

        r0 = sid * rows_per_t
        pltpu.sync_copy(
            table_sh.at[pl.ds(r0, rows_per_t)],
            s_out.at[pl.ds(cid * n + r0, rows_per_t)],
        )
        pltpu.sync_copy(
            cnt_sh.at[pl.ds(r0, rows_per_t)],
            c_out.at[pl.ds(cid * n + r0, rows_per_t)],
        )

    return k(msg, dst)


def _tc_finalize(s_part, c_part, x, root, biasr, ln_wr, ln_br, n):
    hidden = s_part.shape[1]

    def body(s_ref, c_ref, x_ref, root_ref, bias_ref, lnw_ref, lnb_ref, g_ref):
        s = s_ref[:n, :] + s_ref[n:, :]
        cnt = c_ref[:n, 0:1] + c_ref[n:, 0:1]
        aggr = s / jnp.maximum(cnt, 1.0)
        out = aggr + jnp.dot(
            x_ref[...], root_ref[...], preferred_element_type=jnp.float32
        ) + bias_ref[...]
        out = jnp.maximum(out, 0.0)
        mu = jnp.mean(out, axis=-1, keepdims=True)
        var = jnp.mean((out - mu) ** 2, axis=-1, keepdims=True)
        out = (out - mu) / jnp.sqrt(var + 1e-5) * lnw_ref[...] + lnb_ref[...]
        g_ref[...] = jnp.mean(out, axis=0, keepdims=True)

    return pl.pallas_call(
        body,
        out_shape=jax.ShapeDtypeStruct((1, hidden), jnp.float32),
    )(s_part, c_part, x, root, biasr, ln_wr, ln_br)


def kernel(x, edge_index, edge_attr, w1, b1, w2, b2, root, bias, ln_w, ln_b):
    n, node_in = x.shape
    e = edge_attr.shape[0]
    hidden = root.shape[1]

    src = edge_index[0]
    dst = edge_index[1]

    w2p = w2.reshape(hidden, node_in, hidden).transpose(1, 0, 2).reshape(
        node_in * hidden, hidden
    )
    b2r = b2.reshape(node_in, hidden)
    b1r = b1.reshape(1, hidden)

    xj = _sc_gather(x, src)
    msg = _tc_dense(edge_attr, xj, w1, b1r, w2p, b2r, block_e=4000)
    s_part, c_part = _sc_scatter(msg, dst, n)
    g = _tc_finalize(
        s_part,
        c_part,
        x,
        root,
        bias.reshape(1, hidden),
        ln_w.reshape(1, hidden),
        ln_b.reshape(1, hidden),
        n,
    )
    return g


# speedup vs baseline: 3.0007x; 3.0007x over previous
import functools

import jax
import jax.numpy as jnp
from jax import lax
from jax.experimental import pallas as pl
from jax.experimental.pallas import tpu as pltpu
from jax.experimental.pallas import tpu_sc as plsc

_NC = 2
_NS = 16
_NW = _NC * _NS


def _sc_gather(x, src):
    n, node_in = x.shape
    e = src.shape[0]
    e_per_w = e // _NW
    chunk = 80
    n_chunks = e_per_w // chunk
    mesh = plsc.VectorSubcoreMesh(core_axis_name="c", subcore_axis_name="s")

    @functools.partial(
        pl.kernel,
        mesh=mesh,
        out_type=jax.ShapeDtypeStruct((e, node_in), jnp.float32),
        scratch_types=[
            pltpu.VMEM((chunk,), jnp.int32),
            pltpu.VMEM((chunk, node_in), jnp.float32),
            pltpu.VMEM_SHARED((n, node_in), jnp.float32),
            pltpu.SemaphoreType.DMA,
        ],
        compiler_params=pltpu.CompilerParams(use_tc_tiling_on_sc=False),
    )
    def k(x_hbm, src_hbm, out_hbm, idx_v, row_v, x_sh, sem):
        sid = lax.axis_index("s")
        wid = sid * _NC + lax.axis_index("c")
        base = wid * e_per_w

        @pl.when(sid == 0)
        def _():
            pltpu.sync_copy(x_hbm, x_sh)

        plsc.subcore_barrier()

        def chunk_body(ci, _):
            off = base + ci * chunk
            pltpu.sync_copy(src_hbm.at[pl.ds(off, chunk)], idx_v)
            pltpu.async_copy(x_sh.at[idx_v], row_v, sem).wait()
            pltpu.sync_copy(row_v, out_hbm.at[pl.ds(off, chunk)])
            return 0

        lax.fori_loop(0, n_chunks, chunk_body, 0)

    return k(x, src)


def _tc_dense(edge_attr, xj, w1, b1r, w2p, b2r, block_e):
    e, node_in = edge_attr.shape
    hidden = w2p.shape[1]
    grid = e // block_e

    def body(ea_ref, xj_ref, w1_ref, b1_ref, w2p_ref, b2r_ref, out_ref):
        ea = ea_ref[...]
        xj = xj_ref[...][:, :node_in]
        h = jnp.dot(ea, w1_ref[...], preferred_element_type=jnp.float32)
        h = jnp.maximum(h + b1_ref[...], 0.0)
        z = jnp.concatenate(
            [xj[:, i : i + 1] * h for i in range(node_in)], axis=1
        )
        msg = jnp.dot(z, w2p_ref[...], preferred_element_type=jnp.float32)
        msg = msg + jnp.dot(xj, b2r_ref[...], preferred_element_type=jnp.float32)
        out_ref[...] = msg

    return pl.pallas_call(
        body,
        grid=(grid,),
        in_specs=[
            pl.BlockSpec((block_e, node_in), lambda i: (i, 0)),
            pl.BlockSpec((block_e, xj.shape[1]), lambda i: (i, 0)),
            pl.BlockSpec(w1.shape, lambda i: (0, 0)),
            pl.BlockSpec(b1r.shape, lambda i: (0, 0)),
            pl.BlockSpec(w2p.shape, lambda i: (0, 0)),
            pl.BlockSpec(b2r.shape, lambda i: (0, 0)),
        ],
        out_specs=pl.BlockSpec((block_e, hidden), lambda i: (i, 0)),
        out_shape=jax.ShapeDtypeStruct((e, hidden), jnp.float32),
    )(edge_attr, xj, w1, b1r, w2p, b2r)


def _sc_scatter(msg, dst, n):
    e, hidden = msg.shape
    e_per_w = e // _NW
    chunk = 80
    n_chunks = e_per_w // chunk
    n_pad = ((n + 127) // 128) * 128
    rows_per_t = n_pad // _NS
    mesh = plsc.VectorSubcoreMesh(core_axis_name="c", subcore_axis_name="s")

    @functools.partial(
        pl.kernel,
        mesh=mesh,
        out_type=(
            jax.ShapeDtypeStruct((_NC * n_pad, hidden), jnp.float32),
            jax.ShapeDtypeStruct((_NC * n_pad, 16), jnp.float32),
        ),
        scratch_types=[
            pltpu.VMEM((chunk, hidden), jnp.float32),
            pltpu.VMEM((chunk,), jnp.int32),
            pltpu.VMEM((chunk, 16), jnp.float32),
            pltpu.VMEM_SHARED((n_pad, hidden), jnp.float32),
            pltpu.VMEM_SHARED((n_pad, 16), jnp.float32),
        ],
        compiler_params=pltpu.CompilerParams(use_tc_tiling_on_sc=False),
    )
    def k(msg_hbm, dst_hbm, zt_hbm, zc_hbm, one_hbm, s_out, c_out, msg_v,
          idx_v, ones_v, table_sh, cnt_sh):
        cid = lax.axis_index("c")
        sid = lax.axis_index("s")
        wid = sid * _NC + cid

        r0 = sid * rows_per_t

        pltpu.sync_copy(
            zt_hbm.at[pl.ds(r0, rows_per_t)],
            table_sh.at[pl.ds(r0, rows_per_t)],
        )
        pltpu.sync_copy(
            zc_hbm.at[pl.ds(r0, rows_per_t)],
            cnt_sh.at[pl.ds(r0, rows_per_t)],
        )
        pltpu.sync_copy(one_hbm, ones_v)
        plsc.subcore_barrier()

        base = wid * e_per_w

        def chunk_body(ci, _):
            off = base + ci * chunk
            pltpu.sync_copy(msg_hbm.at[pl.ds(off, chunk)], msg_v)
            pltpu.sync_copy(dst_hbm.at[pl.ds(off, chunk)], idx_v)
            pltpu.sync_copy(msg_v, table_sh.at[idx_v], add=True)
            pltpu.sync_copy(ones_v, cnt_sh.at[idx_v], add=True)
            return 0

        lax.fori_loop(0, n_chunks, chunk_body, 0)
        plsc.subcore_barrier()

        r0 = sid * rows_per_t
        pltpu.sync_copy(
            table_sh.at[pl.ds(r0, rows_per_t)],
            s_out.at[pl.ds(cid * n_pad + r0, rows_per_t)],
        )
        pltpu.sync_copy(
            cnt_sh.at[pl.ds(r0, rows_per_t)],
            c_out.at[pl.ds(cid * n_pad + r0, rows_per_t)],
        )

    return k(
        msg,
        dst,
        jnp.zeros((n_pad, hidden), jnp.float32),
        jnp.zeros((n_pad, 16), jnp.float32),
        jnp.ones((chunk, 16), jnp.float32),
    )


def _tc_finalize(s_part, c_part, x, root, biasr, ln_wr, ln_br, n):
    hidden = s_part.shape[1]
    n_pad = s_part.shape[0] // 2

    def body(s_ref, c_ref, x_ref, root_ref, bias_ref, lnw_ref, lnb_ref, g_ref):
        s = s_ref[:n, :] + s_ref[n_pad : n_pad + n, :]
        cnt = c_ref[:n, 0:1] + c_ref[n_pad : n_pad + n, 0:1]
        aggr = s / jnp.maximum(cnt, 1.0)
        out = aggr + jnp.dot(
            x_ref[...], root_ref[...], preferred_element_type=jnp.float32
        ) + bias_ref[...]
        out = jnp.maximum(out, 0.0)
        mu = jnp.mean(out, axis=-1, keepdims=True)
        var = jnp.mean((out - mu) ** 2, axis=-1, keepdims=True)
        out = (out - mu) / jnp.sqrt(var + 1e-5) * lnw_ref[...] + lnb_ref[...]
        g_ref[...] = jnp.mean(out, axis=0, keepdims=True)

    return pl.pallas_call(
        body,
        out_shape=jax.ShapeDtypeStruct((1, hidden), jnp.float32),
    )(s_part, c_part, x, root, biasr, ln_wr, ln_br)


def kernel(x, edge_index, edge_attr, w1, b1, w2, b2, root, bias, ln_w, ln_b):
    n, node_in = x.shape
    e = edge_attr.shape[0]
    hidden = root.shape[1]

    src = edge_index[0]
    dst = edge_index[1]

    w2p = w2.reshape(hidden, node_in, hidden).transpose(1, 0, 2).reshape(
        node_in * hidden, hidden
    )
    b2r = b2.reshape(node_in, hidden)
    b1r = b1.reshape(1, hidden)

    x8 = jnp.pad(x, ((0, 0), (0, 8 - node_in)))
    xj = _sc_gather(x8, src)
    msg = _tc_dense(edge_attr, xj, w1, b1r, w2p, b2r, block_e=4000)
    s_part, c_part = _sc_scatter(msg, dst, n)
    g = _tc_finalize(
        s_part,
        c_part,
        x,
        root,
        bias.reshape(1, hidden),
        ln_w.reshape(1, hidden),
        ln_b.reshape(1, hidden),
        n,
    )
    return g


# speedup vs baseline: 3.5223x; 1.1738x over previous
import functools

import jax
import jax.numpy as jnp
from jax import lax
from jax.experimental import pallas as pl
from jax.experimental.pallas import tpu as pltpu
from jax.experimental.pallas import tpu_sc as plsc

_NC = 2
_NS = 16
_NW = _NC * _NS


def _sc_gather(x, src2):
    n, node_in = x.shape
    rows, il = src2.shape
    e = rows * il
    e_per_w = e // _NW
    sup = 2000
    n_sup = e_per_w // sup
    spr = sup // il
    mesh = plsc.VectorSubcoreMesh(core_axis_name="c", subcore_axis_name="s")

    @functools.partial(
        pl.kernel,
        mesh=mesh,
        out_type=jax.ShapeDtypeStruct((e, node_in), jnp.float32),
        scratch_types=[
            pltpu.VMEM((spr, il), jnp.int32),
            pltpu.VMEM((sup, node_in), jnp.float32),
            pltpu.VMEM_SHARED((n, node_in), jnp.float32),
            pltpu.SemaphoreType.DMA,
        ],
        compiler_params=pltpu.CompilerParams(use_tc_tiling_on_sc=False),
    )
    def k(x_hbm, src_hbm, out_hbm, idx_v, rows_v, x_sh, sem):
        sid = lax.axis_index("s")
        wid = sid * _NC + lax.axis_index("c")
        base = wid * e_per_w

        @pl.when(sid == 0)
        def _():
            pltpu.sync_copy(x_hbm, x_sh)

        plsc.subcore_barrier()

        def sup_body(cj, _):
            off = base + cj * sup
            pltpu.sync_copy(
                src_hbm.at[pl.ds(wid * (e_per_w // il) + cj * spr, spr)],
                idx_v,
            )
            copies = [
                pltpu.async_copy(
                    x_sh.at[idx_v.at[k]],
                    rows_v.at[pl.ds(k * il, il)],
                    sem,
                )
                for k in range(spr)
            ]
            for c in copies:
                c.wait()
            pltpu.sync_copy(rows_v, out_hbm.at[pl.ds(off, sup)])
            return 0

        lax.fori_loop(0, n_sup, sup_body, 0)

    return k(x, src2)


def _tc_dense(edge_attr, xj, w1, b1r, w2p, b2r, block_e):
    e, node_in = edge_attr.shape
    hidden = w2p.shape[1]
    grid = e // block_e

    def body(ea_ref, xj_ref, w1_ref, b1_ref, w2p_ref, b2r_ref, out_ref):
        ea = ea_ref[...]
        xj = xj_ref[...][:, :node_in]
        h = jnp.dot(ea, w1_ref[...], preferred_element_type=jnp.float32)
        h = jnp.maximum(h + b1_ref[...], 0.0)
        z = jnp.concatenate(
            [xj[:, i : i + 1] * h for i in range(node_in)], axis=1
        )
        msg = jnp.dot(z, w2p_ref[...], preferred_element_type=jnp.float32)
        msg = msg + jnp.dot(xj, b2r_ref[...], preferred_element_type=jnp.float32)
        out_ref[...] = msg

    return pl.pallas_call(
        body,
        grid=(grid,),
        in_specs=[
            pl.BlockSpec((block_e, node_in), lambda i: (i, 0)),
            pl.BlockSpec((block_e, xj.shape[1]), lambda i: (i, 0)),
            pl.BlockSpec(w1.shape, lambda i: (0, 0)),
            pl.BlockSpec(b1r.shape, lambda i: (0, 0)),
            pl.BlockSpec(w2p.shape, lambda i: (0, 0)),
            pl.BlockSpec(b2r.shape, lambda i: (0, 0)),
        ],
        out_specs=pl.BlockSpec((block_e, hidden), lambda i: (i, 0)),
        out_shape=jax.ShapeDtypeStruct((e, hidden), jnp.float32),
    )(edge_attr, xj, w1, b1r, w2p, b2r)


def _sc_scatter(msg, dst2, n):
    e, hidden = msg.shape
    rows, il = dst2.shape
    e_per_w = e // _NW
    sup = 400
    spr = sup // il
    n_sup = e_per_w // sup
    n_pad = ((n + 127) // 128) * 128
    rows_per_t = n_pad // _NS
    mesh = plsc.VectorSubcoreMesh(core_axis_name="c", subcore_axis_name="s")

    @functools.partial(
        pl.kernel,
        mesh=mesh,
        out_type=(
            jax.ShapeDtypeStruct((_NC * n_pad, hidden), jnp.float32),
            jax.ShapeDtypeStruct((_NC * n_pad, 16), jnp.float32),
        ),
        scratch_types=[
            pltpu.VMEM((sup, hidden), jnp.float32),
            pltpu.VMEM((spr, il), jnp.int32),
            pltpu.VMEM((il, 16), jnp.float32),
            pltpu.VMEM_SHARED((n_pad, hidden), jnp.float32),
            pltpu.VMEM_SHARED((n_pad, 16), jnp.float32),
            pltpu.SemaphoreType.DMA,
        ],
        compiler_params=pltpu.CompilerParams(use_tc_tiling_on_sc=False),
    )
    def k(msg_hbm, dst_hbm, zt_hbm, zc_hbm, one_hbm, s_out, c_out, msg_v,
          idx_v, ones_v, table_sh, cnt_sh, sem):
        cid = lax.axis_index("c")
        sid = lax.axis_index("s")
        wid = sid * _NC + cid

        r0 = sid * rows_per_t

        pltpu.sync_copy(
            zt_hbm.at[pl.ds(r0, rows_per_t)],
            table_sh.at[pl.ds(r0, rows_per_t)],
        )
        pltpu.sync_copy(
            zc_hbm.at[pl.ds(r0, rows_per_t)],
            cnt_sh.at[pl.ds(r0, rows_per_t)],
        )
        pltpu.sync_copy(one_hbm, ones_v)
        plsc.subcore_barrier()

        base = wid * e_per_w

        def sup_body(cj, _):
            off = base + cj * sup
            pltpu.sync_copy(msg_hbm.at[pl.ds(off, sup)], msg_v)
            pltpu.sync_copy(
                dst_hbm.at[pl.ds(wid * (e_per_w // il) + cj * spr, spr)],
                idx_v,
            )
            copies = []
            for k2 in range(spr):
                copies.append(
                    pltpu.async_copy(
                        msg_v.at[pl.ds(k2 * il, il)],
                        table_sh.at[idx_v.at[k2]],
                        sem,
                        add=True,
                    )
                )
                copies.append(
                    pltpu.async_copy(
                        ones_v, cnt_sh.at[idx_v.at[k2]], sem, add=True
                    )
                )
            for c in copies:
                c.wait()
            return 0

        lax.fori_loop(0, n_sup, sup_body, 0)
        plsc.subcore_barrier()

        r0 = sid * rows_per_t
        pltpu.sync_copy(
            table_sh.at[pl.ds(r0, rows_per_t)],
            s_out.at[pl.ds(cid * n_pad + r0, rows_per_t)],
        )
        pltpu.sync_copy(
            cnt_sh.at[pl.ds(r0, rows_per_t)],
            c_out.at[pl.ds(cid * n_pad + r0, rows_per_t)],
        )

    return k(
        msg,
        dst2,
        jnp.zeros((n_pad, hidden), jnp.float32),
        jnp.zeros((n_pad, 16), jnp.float32),
        jnp.ones((il, 16), jnp.float32),
    )


def _tc_finalize(s_part, c_part, x, root, biasr, ln_wr, ln_br, n):
    hidden = s_part.shape[1]
    n_pad = s_part.shape[0] // 2

    def body(s_ref, c_ref, x_ref, root_ref, bias_ref, lnw_ref, lnb_ref, g_ref):
        s = s_ref[:n, :] + s_ref[n_pad : n_pad + n, :]
        cnt = c_ref[:n, 0:1] + c_ref[n_pad : n_pad + n, 0:1]
        aggr = s / jnp.maximum(cnt, 1.0)
        out = aggr + jnp.dot(
            x_ref[...], root_ref[...], preferred_element_type=jnp.float32
        ) + bias_ref[...]
        out = jnp.maximum(out, 0.0)
        mu = jnp.mean(out, axis=-1, keepdims=True)
        var = jnp.mean((out - mu) ** 2, axis=-1, keepdims=True)
        out = (out - mu) / jnp.sqrt(var + 1e-5) * lnw_ref[...] + lnb_ref[...]
        g_ref[...] = jnp.mean(out, axis=0, keepdims=True)

    return pl.pallas_call(
        body,
        out_shape=jax.ShapeDtypeStruct((1, hidden), jnp.float32),
    )(s_part, c_part, x, root, biasr, ln_wr, ln_br)


def kernel(x, edge_index, edge_attr, w1, b1, w2, b2, root, bias, ln_w, ln_b):
    n, node_in = x.shape
    e = edge_attr.shape[0]
    hidden = root.shape[1]

    src = edge_index[0]
    dst = edge_index[1]

    w2p = w2.reshape(hidden, node_in, hidden).transpose(1, 0, 2).reshape(
        node_in * hidden, hidden
    )
    b2r = b2.reshape(node_in, hidden)
    b1r = b1.reshape(1, hidden)

    x8 = jnp.pad(x, ((0, 0), (0, 8 - node_in)))
    xj = _sc_gather(x8, src.reshape(-1, 80))
    msg = _tc_dense(edge_attr, xj, w1, b1r, w2p, b2r, block_e=4000)
    s_part, c_part = _sc_scatter(msg, dst.reshape(-1, 80), n)
    g = _tc_finalize(
        s_part,
        c_part,
        x,
        root,
        bias.reshape(1, hidden),
        ln_w.reshape(1, hidden),
        ln_b.reshape(1, hidden),
        n,
    )
    return g


# speedup vs baseline: 3.5804x; 1.0165x over previous
import functools

import jax
import jax.numpy as jnp
from jax import lax
from jax.experimental import pallas as pl
from jax.experimental.pallas import tpu as pltpu
from jax.experimental.pallas import tpu_sc as plsc

_NC = 2
_NS = 16
_NW = _NC * _NS


def _sc_gather(x, src2, dst2, n_pad):
    n, node_in = x.shape
    rows, il = src2.shape
    e = rows * il
    e_per_w = e // _NW
    sup = 2000
    n_sup = e_per_w // sup
    spr = sup // il
    rows_per_t = n_pad // _NS
    mesh = plsc.VectorSubcoreMesh(core_axis_name="c", subcore_axis_name="s")

    @functools.partial(
        pl.kernel,
        mesh=mesh,
        out_type=(
            jax.ShapeDtypeStruct((e, node_in), jnp.float32),
            jax.ShapeDtypeStruct((_NC * n_pad, 16), jnp.float32),
        ),
        scratch_types=[
            pltpu.VMEM((spr, il), jnp.int32),
            pltpu.VMEM((spr, il), jnp.int32),
            pltpu.VMEM((sup, node_in), jnp.float32),
            pltpu.VMEM((il, 16), jnp.float32),
            pltpu.VMEM_SHARED((n, node_in), jnp.float32),
            pltpu.VMEM_SHARED((n_pad, 16), jnp.float32),
            pltpu.SemaphoreType.DMA,
            pltpu.SemaphoreType.DMA,
        ],
        compiler_params=pltpu.CompilerParams(use_tc_tiling_on_sc=False),
    )
    def k(x_hbm, src_hbm, dst_hbm, zc_hbm, one_hbm, out_hbm, c_out, idx_v,
          didx_v, rows_v, ones_v, x_sh, cnt_sh, sem, sem2):
        cid = lax.axis_index("c")
        sid = lax.axis_index("s")
        wid = sid * _NC + cid
        base = wid * e_per_w
        r0 = sid * rows_per_t

        @pl.when(sid == 0)
        def _():
            pltpu.sync_copy(x_hbm, x_sh)

        pltpu.sync_copy(
            zc_hbm.at[pl.ds(r0, rows_per_t)],
            cnt_sh.at[pl.ds(r0, rows_per_t)],
        )
        pltpu.sync_copy(one_hbm, ones_v)
        plsc.subcore_barrier()

        def sup_body(cj, _):
            off = base + cj * sup
            ro = wid * (e_per_w // il) + cj * spr
            pltpu.sync_copy(src_hbm.at[pl.ds(ro, spr)], idx_v)
            pltpu.sync_copy(dst_hbm.at[pl.ds(ro, spr)], didx_v)
            copies = []
            for k2 in range(spr):
                copies.append(
                    pltpu.async_copy(
                        x_sh.at[idx_v.at[k2]],
                        rows_v.at[pl.ds(k2 * il, il)],
                        sem,
                    )
                )
                copies.append(
                    pltpu.async_copy(
                        ones_v, cnt_sh.at[didx_v.at[k2]], sem2, add=True
                    )
                )
            for c in copies:
                c.wait()
            pltpu.sync_copy(rows_v, out_hbm.at[pl.ds(off, sup)])
            return 0

        lax.fori_loop(0, n_sup, sup_body, 0)
        plsc.subcore_barrier()
        pltpu.sync_copy(
            cnt_sh.at[pl.ds(r0, rows_per_t)],
            c_out.at[pl.ds(cid * n_pad + r0, rows_per_t)],
        )

    return k(
        x,
        src2,
        dst2,
        jnp.zeros((n_pad, 16), jnp.float32),
        jnp.ones((il, 16), jnp.float32),
    )


def _tc_dense(edge_attr, xj, w1, b1r, w2p, b2r, block_e):
    e, node_in = edge_attr.shape
    hidden = w2p.shape[1]
    grid = e // block_e

    def body(ea_ref, xj_ref, w1_ref, b1_ref, w2p_ref, b2r_ref, out_ref):
        ea = ea_ref[...]
        xj = xj_ref[...][:, :node_in]
        h = jnp.dot(ea, w1_ref[...], preferred_element_type=jnp.float32)
        h = jnp.maximum(h + b1_ref[...], 0.0)
        z = jnp.concatenate(
            [xj[:, i : i + 1] * h for i in range(node_in)], axis=1
        )
        msg = jnp.dot(z, w2p_ref[...], preferred_element_type=jnp.float32)
        msg = msg + jnp.dot(xj, b2r_ref[...], preferred_element_type=jnp.float32)
        out_ref[...] = msg

    return pl.pallas_call(
        body,
        grid=(grid,),
        in_specs=[
            pl.BlockSpec((block_e, node_in), lambda i: (i, 0)),
            pl.BlockSpec((block_e, xj.shape[1]), lambda i: (i, 0)),
            pl.BlockSpec(w1.shape, lambda i: (0, 0)),
            pl.BlockSpec(b1r.shape, lambda i: (0, 0)),
            pl.BlockSpec(w2p.shape, lambda i: (0, 0)),
            pl.BlockSpec(b2r.shape, lambda i: (0, 0)),
        ],
        out_specs=pl.BlockSpec((block_e, hidden), lambda i: (i, 0)),
        out_shape=jax.ShapeDtypeStruct((e, hidden), jnp.float32),
    )(edge_attr, xj, w1, b1r, w2p, b2r)


def _sc_scatter(msg, dst2, n):
    e, hidden = msg.shape
    rows, il = dst2.shape
    e_per_w = e // _NW
    sup = 400
    spr = sup // il
    n_sup = e_per_w // sup
    n_pad = ((n + 127) // 128) * 128
    rows_per_t = n_pad // _NS
    mesh = plsc.VectorSubcoreMesh(core_axis_name="c", subcore_axis_name="s")

    @functools.partial(
        pl.kernel,
        mesh=mesh,
        out_type=jax.ShapeDtypeStruct((_NC * n_pad, hidden), jnp.float32),
        scratch_types=[
            pltpu.VMEM((sup, hidden), jnp.float32),
            pltpu.VMEM((spr, il), jnp.int32),
            pltpu.VMEM_SHARED((n_pad, hidden), jnp.float32),
            pltpu.SemaphoreType.DMA,
        ],
        compiler_params=pltpu.CompilerParams(use_tc_tiling_on_sc=False),
    )
    def k(msg_hbm, dst_hbm, zt_hbm, s_out, msg_v, idx_v, table_sh, sem):
        cid = lax.axis_index("c")
        sid = lax.axis_index("s")
        wid = sid * _NC + cid

        r0 = sid * rows_per_t

        pltpu.sync_copy(
            zt_hbm.at[pl.ds(r0, rows_per_t)],
            table_sh.at[pl.ds(r0, rows_per_t)],
        )
        plsc.subcore_barrier()

        base = wid * e_per_w

        def sup_body(cj, _):
            off = base + cj * sup
            pltpu.sync_copy(msg_hbm.at[pl.ds(off, sup)], msg_v)
            pltpu.sync_copy(
                dst_hbm.at[pl.ds(wid * (e_per_w // il) + cj * spr, spr)],
                idx_v,
            )
            copies = [
                pltpu.async_copy(
                    msg_v.at[pl.ds(k2 * il, il)],
                    table_sh.at[idx_v.at[k2]],
                    sem,
                    add=True,
                )
                for k2 in range(spr)
            ]
            for c in copies:
                c.wait()
            return 0

        lax.fori_loop(0, n_sup, sup_body, 0)
        plsc.subcore_barrier()

        pltpu.sync_copy(
            table_sh.at[pl.ds(r0, rows_per_t)],
            s_out.at[pl.ds(cid * n_pad + r0, rows_per_t)],
        )

    return k(msg, dst2, jnp.zeros((n_pad, hidden), jnp.float32))


def _tc_finalize(s_part, c_part, x, root, biasr, ln_wr, ln_br, n):
    hidden = s_part.shape[1]
    n_pad = s_part.shape[0] // 2

    def body(s_ref, c_ref, x_ref, root_ref, bias_ref, lnw_ref, lnb_ref, g_ref):
        s = s_ref[:n, :] + s_ref[n_pad : n_pad + n, :]
        cnt = c_ref[:n, 0:1] + c_ref[n_pad : n_pad + n, 0:1]
        aggr = s / jnp.maximum(cnt, 1.0)
        out = aggr + jnp.dot(
            x_ref[...], root_ref[...], preferred_element_type=jnp.float32
        ) + bias_ref[...]
        out = jnp.maximum(out, 0.0)
        mu = jnp.mean(out, axis=-1, keepdims=True)
        var = jnp.mean((out - mu) ** 2, axis=-1, keepdims=True)
        out = (out - mu) / jnp.sqrt(var + 1e-5) * lnw_ref[...] + lnb_ref[...]
        g_ref[...] = jnp.mean(out, axis=0, keepdims=True)

    return pl.pallas_call(
        body,
        out_shape=jax.ShapeDtypeStruct((1, hidden), jnp.float32),
    )(s_part, c_part, x, root, biasr, ln_wr, ln_br)


def kernel(x, edge_index, edge_attr, w1, b1, w2, b2, root, bias, ln_w, ln_b):
    n, node_in = x.shape
    e = edge_attr.shape[0]
    hidden = root.shape[1]

    src = edge_index[0]
    dst = edge_index[1]

    w2p = w2.reshape(hidden, node_in, hidden).transpose(1, 0, 2).reshape(
        node_in * hidden, hidden
    )
    b2r = b2.reshape(node_in, hidden)
    b1r = b1.reshape(1, hidden)

    x8 = jnp.pad(x, ((0, 0), (0, 8 - node_in)))
    n_pad = ((n + 127) // 128) * 128
    xj, c_part = _sc_gather(x8, src.reshape(-1, 80), dst.reshape(-1, 80), n_pad)
    msg = _tc_dense(edge_attr, xj, w1, b1r, w2p, b2r, block_e=4000)
    s_part = _sc_scatter(msg, dst.reshape(-1, 80), n)
    g = _tc_finalize(
        s_part,
        c_part,
        x,
        root,
        bias.reshape(1, hidden),
        ln_w.reshape(1, hidden),
        ln_b.reshape(1, hidden),
        n,
    )
    return g


# speedup vs baseline: 3.5919x; 1.0032x over previous
import functools

import jax
import jax.numpy as jnp
from jax import lax
from jax.experimental import pallas as pl
from jax.experimental.pallas import tpu as pltpu
from jax.experimental.pallas import tpu_sc as plsc

_NC = 2
_NS = 16
_NW = _NC * _NS


def _sc_gather(x, src2, dst2, n_pad):
    n, node_in = x.shape
    rows, il = src2.shape
    e = rows * il
    e_per_w = e // _NW
    sup = 2000
    n_sup = e_per_w // sup
    spr = sup // il
    rows_per_t = n_pad // _NS
    mesh = plsc.VectorSubcoreMesh(core_axis_name="c", subcore_axis_name="s")

    @functools.partial(
        pl.kernel,
        mesh=mesh,
        out_type=(
            jax.ShapeDtypeStruct((e, node_in), jnp.float32),
            jax.ShapeDtypeStruct((_NC * n_pad, 16), jnp.float32),
        ),
        scratch_types=[
            pltpu.VMEM((spr, il), jnp.int32),
            pltpu.VMEM((spr, il), jnp.int32),
            pltpu.VMEM((sup, node_in), jnp.float32),
            pltpu.VMEM((il, 16), jnp.float32),
            pltpu.VMEM_SHARED((n, node_in), jnp.float32),
            pltpu.VMEM_SHARED((n_pad, 16), jnp.float32),
            pltpu.SemaphoreType.DMA,
            pltpu.SemaphoreType.DMA,
        ],
        compiler_params=pltpu.CompilerParams(use_tc_tiling_on_sc=False),
    )
    def k(x_hbm, src_hbm, dst_hbm, zc_hbm, one_hbm, out_hbm, c_out, idx_v,
          didx_v, rows_v, ones_v, x_sh, cnt_sh, sem, sem2):
        cid = lax.axis_index("c")
        sid = lax.axis_index("s")
        wid = sid * _NC + cid
        base = wid * e_per_w
        r0 = sid * rows_per_t

        @pl.when(sid == 0)
        def _():
            pltpu.sync_copy(x_hbm, x_sh)

        pltpu.sync_copy(
            zc_hbm.at[pl.ds(r0, rows_per_t)],
            cnt_sh.at[pl.ds(r0, rows_per_t)],
        )
        pltpu.sync_copy(one_hbm, ones_v)
        plsc.subcore_barrier()

        def sup_body(cj, _):
            off = base + cj * sup
            ro = wid * (e_per_w // il) + cj * spr
            pltpu.sync_copy(src_hbm.at[pl.ds(ro, spr)], idx_v)
            pltpu.sync_copy(dst_hbm.at[pl.ds(ro, spr)], didx_v)
            copies = []
            for k2 in range(spr):
                copies.append(
                    pltpu.async_copy(
                        x_sh.at[idx_v.at[k2]],
                        rows_v.at[pl.ds(k2 * il, il)],
                        sem,
                    )
                )
                copies.append(
                    pltpu.async_copy(
                        ones_v, cnt_sh.at[didx_v.at[k2]], sem2, add=True
                    )
                )
            for c in copies:
                c.wait()
            pltpu.sync_copy(rows_v, out_hbm.at[pl.ds(off, sup)])
            return 0

        lax.fori_loop(0, n_sup, sup_body, 0)
        plsc.subcore_barrier()
        pltpu.sync_copy(
            cnt_sh.at[pl.ds(r0, rows_per_t)],
            c_out.at[pl.ds(cid * n_pad + r0, rows_per_t)],
        )

    return k(
        x,
        src2,
        dst2,
        jnp.zeros((n_pad, 16), jnp.float32),
        jnp.ones((il, 16), jnp.float32),
    )


def _tc_dense(edge_attr, xj, w1, b1r, w2p, b2r, block_e):
    e, node_in = edge_attr.shape
    hidden = w2p.shape[1]
    grid = e // block_e

    def body(ea_ref, xj_ref, w1_ref, b1_ref, w2p_ref, b2r_ref, out_ref):
        ea = ea_ref[...]
        xj = xj_ref[...][:, :node_in]
        h = jnp.dot(ea, w1_ref[...], preferred_element_type=jnp.float32)
        h = jnp.maximum(h + b1_ref[...], 0.0)
        z = jnp.concatenate(
            [xj[:, i : i + 1] * h for i in range(node_in)], axis=1
        )
        msg = jnp.dot(z, w2p_ref[...], preferred_element_type=jnp.float32)
        msg = msg + jnp.dot(xj, b2r_ref[...], preferred_element_type=jnp.float32)
        out_ref[...] = msg

    return pl.pallas_call(
        body,
        grid=(grid,),
        in_specs=[
            pl.BlockSpec((block_e, node_in), lambda i: (i, 0)),
            pl.BlockSpec((block_e, xj.shape[1]), lambda i: (i, 0)),
            pl.BlockSpec(w1.shape, lambda i: (0, 0)),
            pl.BlockSpec(b1r.shape, lambda i: (0, 0)),
            pl.BlockSpec(w2p.shape, lambda i: (0, 0)),
            pl.BlockSpec(b2r.shape, lambda i: (0, 0)),
        ],
        out_specs=pl.BlockSpec((block_e, hidden), lambda i: (i, 0)),
        out_shape=jax.ShapeDtypeStruct((e, hidden), jnp.float32),
    )(edge_attr, xj, w1, b1r, w2p, b2r)


def _sc_scatter(msg, dst2, n):
    e, hidden = msg.shape
    rows, il = dst2.shape
    e_per_w = e // _NW
    sup = 400
    spr = sup // il
    n_sup = e_per_w // sup
    n_pad = ((n + 127) // 128) * 128
    rows_per_t = n_pad // _NS
    mesh = plsc.VectorSubcoreMesh(core_axis_name="c", subcore_axis_name="s")

    @functools.partial(
        pl.kernel,
        mesh=mesh,
        out_type=jax.ShapeDtypeStruct((_NC * n_pad, hidden), jnp.float32),
        scratch_types=[
            pltpu.VMEM((sup, hidden), jnp.float32),
            pltpu.VMEM((spr, il), jnp.int32),
            pltpu.VMEM_SHARED((n_pad, hidden), jnp.float32),
            pltpu.SemaphoreType.DMA,
        ],
        compiler_params=pltpu.CompilerParams(use_tc_tiling_on_sc=False),
    )
    def k(msg_hbm, dst_hbm, zt_hbm, s_out, msg_v, idx_v, table_sh, sem):
        cid = lax.axis_index("c")
        sid = lax.axis_index("s")
        wid = sid * _NC + cid

        r0 = sid * rows_per_t

        pltpu.sync_copy(
            zt_hbm.at[pl.ds(r0, rows_per_t)],
            table_sh.at[pl.ds(r0, rows_per_t)],
        )
        plsc.subcore_barrier()

        base = wid * e_per_w

        def sup_body(cj, _):
            off = base + cj * sup
            pltpu.sync_copy(msg_hbm.at[pl.ds(off, sup)], msg_v)
            pltpu.sync_copy(
                dst_hbm.at[pl.ds(wid * (e_per_w // il) + cj * spr, spr)],
                idx_v,
            )
            copies = [
                pltpu.async_copy(
                    msg_v.at[pl.ds(k2 * il, il)],
                    table_sh.at[idx_v.at[k2]],
                    sem,
                    add=True,
                )
                for k2 in range(spr)
            ]
            for c in copies:
                c.wait()
            return 0

        lax.fori_loop(0, n_sup, sup_body, 0)
        plsc.subcore_barrier()

        pltpu.sync_copy(
            table_sh.at[pl.ds(r0, rows_per_t)],
            s_out.at[pl.ds(cid * n_pad + r0, rows_per_t)],
        )

    return k(msg, dst2, jnp.zeros((n_pad, hidden), jnp.float32))


def _tc_finalize(s_part, c_part, x, root, biasr, ln_wr, ln_br, n):
    hidden = s_part.shape[1]
    n_pad = s_part.shape[0] // 2

    def body(s_ref, c_ref, x_ref, root_ref, bias_ref, lnw_ref, lnb_ref, g_ref):
        s = s_ref[:n, :] + s_ref[n_pad : n_pad + n, :]
        cnt = c_ref[:n, 0:1] + c_ref[n_pad : n_pad + n, 0:1]
        aggr = s / jnp.maximum(cnt, 1.0)
        out = aggr + jnp.dot(
            x_ref[...], root_ref[...], preferred_element_type=jnp.float32
        ) + bias_ref[...]
        out = jnp.maximum(out, 0.0)
        mu = jnp.mean(out, axis=-1, keepdims=True)
        var = jnp.mean((out - mu) ** 2, axis=-1, keepdims=True)
        out = (out - mu) / jnp.sqrt(var + 1e-5) * lnw_ref[...] + lnb_ref[...]
        g_ref[...] = jnp.mean(out, axis=0, keepdims=True)

    return pl.pallas_call(
        body,
        out_shape=jax.ShapeDtypeStruct((1, hidden), jnp.float32),
    )(s_part, c_part, x, root, biasr, ln_wr, ln_br)


def kernel(x, edge_index, edge_attr, w1, b1, w2, b2, root, bias, ln_w, ln_b):
    n, node_in = x.shape
    e = edge_attr.shape[0]
    hidden = root.shape[1]

    src = edge_index[0]
    dst = edge_index[1]

    w2p = w2.reshape(hidden, node_in, hidden).transpose(1, 0, 2).reshape(
        node_in * hidden, hidden
    )
    b2r = b2.reshape(node_in, hidden)
    b1r = b1.reshape(1, hidden)

    x8 = jnp.pad(x, ((0, 0), (0, 8 - node_in)))
    n_pad = ((n + 127) // 128) * 128
    xj, c_part = _sc_gather(x8, src.reshape(-1, 80), dst.reshape(-1, 80), n_pad)
    msg = _tc_dense(edge_attr, xj, w1, b1r, w2p, b2r, block_e=8000)
    s_part = _sc_scatter(msg, dst.reshape(-1, 80), n)
    g = _tc_finalize(
        s_part,
        c_part,
        x,
        root,
        bias.reshape(1, hidden),
        ln_w.reshape(1, hidden),
        ln_b.reshape(1, hidden),
        n,
    )
    return g
